# baseline (speedup 1.0000x reference)
import functools
import math

import jax
import jax.numpy as jnp
from jax import lax
from jax.experimental import pallas as pl
from jax.experimental.pallas import tpu as pltpu
from jax.experimental.pallas import tpu_sc as plsc

N = 10000
E = 320000
H = 128
EPS = 1e-5

_NC = 2
_NS = 16
_NW = _NC * _NS
_EPW = E // _NW
_CH = 80
_NCHUNK = _EPW // _CH
_NP = 10240
_RPT = _NP // _NS
_ZR = 64

_INV_BN = 1.0 / math.sqrt(1.0 + EPS)

_mesh = plsc.VectorSubcoreMesh(core_axis_name="c", subcore_axis_name="s")


_ER = E // 128
_FULLC = _ER // _NW
_EXTRA = _ER % _NW


def _sc_layer_body(xs_hbm, ei_hbm, out_hbm, idx, rows, zbuf, acc,
                   sem_i, sem_r):
    c = lax.axis_index("c")
    s = lax.axis_index("s")
    wid = c * _NS + s
    nchunk = _FULLC + jnp.where(wid < _EXTRA, 1, 0)

    zeros16 = jnp.zeros((16,), jnp.float32)

    def zfill(i, _):
        for j in range(H // 16):
            zbuf[i, pl.ds(j * 16, 16)] = zeros16
        return ()
    lax.fori_loop(0, _ZR, zfill, ())

    for k in range(_RPT // _ZR):
        pltpu.sync_copy(zbuf, acc.at[pl.ds(s * _RPT + k * _ZR, _ZR)])
    plsc.subcore_barrier()

    def fire_idx(g, b):
        pltpu.async_copy(ei_hbm.at[wid + _NW * g], idx.at[b], sem_i.at[b])

    def wait_idx(b):
        pltpu.make_async_copy(ei_hbm.at[0], idx.at[b], sem_i.at[b]).wait()

    def fire_gather(b):
        pltpu.async_copy(xs_hbm.at[idx.at[b, 0]], rows.at[b], sem_r.at[b])

    def wait_gather(b):
        pltpu.make_async_copy(xs_hbm.at[idx.at[b, 0]], rows.at[b],
                              sem_r.at[b]).wait()

    fire_idx(0, 0)

    @pl.when(nchunk > 1)
    def _():
        fire_idx(1, 1)
    wait_idx(0)
    fire_gather(0)

    def body(g, _):
        b = lax.rem(g, 2)
        nb = lax.rem(g + 1, 2)

        @pl.when(g + 1 < nchunk)
        def _():
            wait_idx(nb)
            fire_gather(nb)
        wait_gather(b)
        pltpu.sync_copy(rows.at[b], acc.at[idx.at[b, 1]], add=True)

        @pl.when(g + 2 < nchunk)
        def _():
            fire_idx(g + 2, b)
        return ()
    lax.fori_loop(0, nchunk, body, ())

    plsc.subcore_barrier()
    pltpu.sync_copy(acc.at[pl.ds(s * _RPT, _RPT)],
                    out_hbm.at[c].at[pl.ds(s * _RPT, _RPT)])


_sc_layer = pl.kernel(
    _sc_layer_body,
    mesh=_mesh,
    out_type=jax.ShapeDtypeStruct((_NC, _NP, H), jnp.float32),
    scratch_types=[
        pltpu.VMEM((2, 2, 128), jnp.int32),
        pltpu.VMEM((2, 128, H), jnp.float32),
        pltpu.VMEM((_ZR, H), jnp.float32),
        pltpu.VMEM_SHARED((_NP, H), jnp.float32),
        pltpu.SemaphoreType.DMA((2,)),
        pltpu.SemaphoreType.DMA((2,)),
    ],
)


_RB = 1000


def _dinv_block(degp):
    deg = degp[0, :, 0:1] + degp[1, :, 0:1] + 1.0
    return lax.rsqrt(deg)


def _tc_input_body(x, winT, b_in, g_in, be_in, wc0T, degp, h_out, xs_out):
    hv = jnp.dot(x[...], winT[...], preferred_element_type=jnp.float32)
    hv = (hv + b_in[...]) * _INV_BN * g_in[...] + be_in[...]
    hv = jnp.maximum(hv, 0.0)
    h_out[...] = hv
    dinv = _dinv_block(degp[...])
    xs_out[...] = jnp.dot(hv, wc0T[...],
                          preferred_element_type=jnp.float32) * dinv


def _tc_mid_body(h, xs, acc, degp, b, wT, h_out, xs_out):
    dinv = _dinv_block(degp[...])
    tot = acc[0] + acc[1] + xs[...]
    o = tot * dinv + b[...]
    elu = jnp.where(o > 0.0, o, jnp.exp(o) - 1.0)
    hn = h[...] + 0.3 * elu
    h_out[...] = hn
    xs_out[...] = jnp.dot(hn, wT[...],
                          preferred_element_type=jnp.float32) * dinv


def _sigmoid(t):
    return 1.0 / (1.0 + jnp.exp(-t))


def _tc_final_body(h, xs, acc, degp, b,
                   wcl1T, bcl1, g_cl, be_cl, wcl2T, bcl2,
                   wr1T, br1, wr2T, br2, wr3T, br3,
                   wz1T, bz1, wz2T, bz2, wz3T, bz3,
                   th_out, risk_out, zd_out):
    dinv = _dinv_block(degp[...])
    tot = acc[0] + acc[1] + xs[...]
    o = tot * dinv + b[...]
    elu = jnp.where(o > 0.0, o, jnp.exp(o) - 1.0)
    hn = h[...] + 0.3 * elu

    cb = jnp.dot(hn, wcl1T[...], preferred_element_type=jnp.float32)
    cb = (cb + bcl1[...]) * _INV_BN * g_cl[...] + be_cl[...]
    cb = jnp.maximum(cb, 0.0)
    th_out[...] = jnp.dot(cb, wcl2T[...],
                          preferred_element_type=jnp.float32) + bcl2[...]

    r = jnp.maximum(jnp.dot(hn, wr1T[...],
                            preferred_element_type=jnp.float32) + br1[...], 0.0)
    r = jnp.maximum(jnp.dot(r, wr2T[...],
                            preferred_element_type=jnp.float32) + br2[...], 0.0)
    risk_out[...] = _sigmoid(jnp.dot(r, wr3T[...],
                                     preferred_element_type=jnp.float32) + br3[...])

    z = jnp.maximum(jnp.dot(hn, wz1T[...],
                            preferred_element_type=jnp.float32) + bz1[...], 0.0)
    z = jnp.maximum(jnp.dot(z, wz2T[...],
                            preferred_element_type=jnp.float32) + bz2[...], 0.0)
    zd_out[...] = _sigmoid(jnp.dot(z, wz3T[...],
                                   preferred_element_type=jnp.float32) + bz3[...])


def _row_spec(width):
    return pl.BlockSpec((_RB, width), lambda i: (i, 0))


def _full_spec(shape):
    nd = len(shape)
    return pl.BlockSpec(shape, lambda i: (0,) * nd)


_ACC_SPEC = pl.BlockSpec((_NC, _RB, H), lambda i: (0, i, 0))
_DEGP_SPEC = _ACC_SPEC
_GRID = (N // _RB,)


def _tc_input(x, winT, b_in, g_in, be_in, wc0T, degp):
    return pl.pallas_call(
        _tc_input_body,
        grid=_GRID,
        in_specs=[
            _row_spec(H), _full_spec((H, H)), _full_spec((1, H)),
            _full_spec((1, H)), _full_spec((1, H)), _full_spec((H, H)),
            _DEGP_SPEC,
        ],
        out_specs=[_row_spec(H), _row_spec(H)],
        out_shape=[jax.ShapeDtypeStruct((N, H), jnp.float32),
                   jax.ShapeDtypeStruct((N, H), jnp.float32)],
    )(x, winT, b_in, g_in, be_in, wc0T, degp)


def _tc_mid(h, xs, acc, degp, b, wT):
    return pl.pallas_call(
        _tc_mid_body,
        grid=_GRID,
        in_specs=[
            _row_spec(H), _row_spec(H), _ACC_SPEC, _DEGP_SPEC,
            _full_spec((1, H)), _full_spec((H, H)),
        ],
        out_specs=[_row_spec(H), _row_spec(H)],
        out_shape=[jax.ShapeDtypeStruct((N, H), jnp.float32),
                   jax.ShapeDtypeStruct((N, H), jnp.float32)],
    )(h, xs, acc, degp, b, wT)


def _tc_final(h, xs, acc, degp, b, *heads):
    head_specs = [_full_spec(w.shape) for w in heads]
    return pl.pallas_call(
        _tc_final_body,
        grid=_GRID,
        in_specs=[
            _row_spec(H), _row_spec(H), _ACC_SPEC, _DEGP_SPEC,
            _full_spec((1, H)),
        ] + head_specs,
        out_specs=[_row_spec(56), _row_spec(1), _row_spec(1)],
        out_shape=[jax.ShapeDtypeStruct((N, 56), jnp.float32),
                   jax.ShapeDtypeStruct((N, 1), jnp.float32),
                   jax.ShapeDtypeStruct((N, 1), jnp.float32)],
    )(h, xs, acc, degp, b, *heads)


def kernel(x, edge_index, W_in, b_in, g_in, be_in, Wc0, bc0, Wc1, bc1,
           Wc2, bc2, Wcl1, bcl1, g_cl, be_cl, Wcl2, bcl2, Wr1, br1,
           Wr2, br2, Wr3, br3, Wz1, bz1, Wz2, bz2, Wz3, bz3):
    ei = edge_index.reshape(2, _ER, 128).transpose(1, 0, 2)

    ones = jnp.ones((N, H), jnp.float32)
    ei0 = ei.at[:, 0, :].set(0)
    degp = _sc_layer(ones, ei0)

    row = lambda v: v.reshape(1, -1)
    h, xs = _tc_input(x, W_in.T, row(b_in), row(g_in), row(be_in),
                      Wc0.T, degp)

    acc = _sc_layer(xs, ei)
    h, xs = _tc_mid(h, xs, acc, degp, row(bc0), Wc1.T)

    acc = _sc_layer(xs, ei)
    h, xs = _tc_mid(h, xs, acc, degp, row(bc1), Wc2.T)

    acc = _sc_layer(xs, ei)
    threat, risk, zero_day = _tc_final(
        h, xs, acc, degp, row(bc2),
        Wcl1.T, row(bcl1), row(g_cl), row(be_cl), Wcl2.T, row(bcl2),
        Wr1.T, row(br1), Wr2.T, row(br2), Wr3.T, row(br3),
        Wz1.T, row(bz1), Wz2.T, row(bz2), Wz3.T, row(bz3))
    return (threat, risk, zero_day)


# speedup vs baseline: 17.6821x; 17.6821x over previous
import functools
import math

import jax
import jax.numpy as jnp
from jax import lax
from jax.experimental import pallas as pl
from jax.experimental.pallas import tpu as pltpu
from jax.experimental.pallas import tpu_sc as plsc

N = 10000
E = 320000
H = 128
EPS = 1e-5

_NC = 2
_NS = 16
_NW = _NC * _NS
_EPW = E // _NW
_CH = 80
_NCHUNK = _EPW // _CH
_NP = 10240
_RPT = _NP // _NS
_ZR = 64

_INV_BN = 1.0 / math.sqrt(1.0 + EPS)

_mesh = plsc.VectorSubcoreMesh(core_axis_name="c", subcore_axis_name="s")


_ER = E // 128
_FULLC = _ER // _NW
_EXTRA = _ER % _NW


def _sc_layer_body(xs_hbm, ei_hbm, out_hbm, idx, rows, zbuf, acc,
                   sem_i, sem_r):
    c = lax.axis_index("c")
    s = lax.axis_index("s")
    wid = c * _NS + s
    nchunk = _FULLC + jnp.where(wid < _EXTRA, 1, 0)

    zeros16 = jnp.zeros((16,), jnp.float32)

    def zfill(i, _):
        for j in range(H // 16):
            zbuf[i, pl.ds(j * 16, 16)] = zeros16
        return ()
    lax.fori_loop(0, _ZR, zfill, ())

    for k in range(_RPT // _ZR):
        pltpu.sync_copy(zbuf, acc.at[pl.ds(s * _RPT + k * _ZR, _ZR)])
    plsc.subcore_barrier()

    def fire_idx(g, b):
        pltpu.async_copy(ei_hbm.at[wid + _NW * g], idx.at[b], sem_i.at[b])

    def wait_idx(b):
        pltpu.make_async_copy(ei_hbm.at[0], idx.at[b], sem_i.at[b]).wait()

    def fire_gather(b):
        pltpu.async_copy(xs_hbm.at[idx.at[b, 0]], rows.at[b], sem_r.at[b])

    def wait_gather(b):
        pltpu.make_async_copy(xs_hbm.at[idx.at[b, 0]], rows.at[b],
                              sem_r.at[b]).wait()

    fire_idx(0, 0)

    @pl.when(nchunk > 1)
    def _():
        fire_idx(1, 1)
    wait_idx(0)
    fire_gather(0)

    def body(g, _):
        b = lax.rem(g, 2)
        nb = lax.rem(g + 1, 2)

        @pl.when(g + 1 < nchunk)
        def _():
            wait_idx(nb)
            fire_gather(nb)
        wait_gather(b)
        pltpu.sync_copy(rows.at[b], acc.at[idx.at[b, 1]], add=True)

        @pl.when(g + 2 < nchunk)
        def _():
            fire_idx(g + 2, b)
        return ()
    lax.fori_loop(0, nchunk, body, ())

    plsc.subcore_barrier()
    pltpu.sync_copy(acc.at[pl.ds(s * _RPT, _RPT)],
                    out_hbm.at[c].at[pl.ds(s * _RPT, _RPT)])


_sc_layer = pl.kernel(
    _sc_layer_body,
    mesh=_mesh,
    out_type=jax.ShapeDtypeStruct((_NC, _NP, H), jnp.float32),
    scratch_types=[
        pltpu.VMEM((2, 2, 128), jnp.int32),
        pltpu.VMEM((2, 128, H), jnp.float32),
        pltpu.VMEM((_ZR, H), jnp.float32),
        pltpu.VMEM_SHARED((_NP, H), jnp.float32),
        pltpu.SemaphoreType.DMA((2,)),
        pltpu.SemaphoreType.DMA((2,)),
    ],
)


_RB = 1000


def _dinv_block(degp):
    deg = degp[0, :, 0:1] + degp[1, :, 0:1] + 1.0
    return lax.rsqrt(deg)


def _tc_input_body(x, winT, b_in, g_in, be_in, wc0T, degp, h_out, xs_out):
    hv = jnp.dot(x[...], winT[...], preferred_element_type=jnp.float32)
    hv = (hv + b_in[...]) * _INV_BN * g_in[...] + be_in[...]
    hv = jnp.maximum(hv, 0.0)
    h_out[...] = hv
    dinv = _dinv_block(degp[...])
    xs_out[...] = jnp.dot(hv, wc0T[...],
                          preferred_element_type=jnp.float32) * dinv


def _tc_mid_body(h, xs, acc, degp, b, wT, h_out, xs_out):
    dinv = _dinv_block(degp[...])
    tot = acc[0] + acc[1] + xs[...]
    o = tot * dinv + b[...]
    elu = jnp.where(o > 0.0, o, jnp.exp(o) - 1.0)
    hn = h[...] + 0.3 * elu
    h_out[...] = hn
    xs_out[...] = jnp.dot(hn, wT[...],
                          preferred_element_type=jnp.float32) * dinv


def _sigmoid(t):
    return 1.0 / (1.0 + jnp.exp(-t))


def _tc_final_body(h, xs, acc, degp, b,
                   wcl1T, bcl1, g_cl, be_cl, wcl2T, bcl2,
                   wr1T, br1, wr2T, br2, wr3T, br3,
                   wz1T, bz1, wz2T, bz2, wz3T, bz3,
                   th_out, risk_out, zd_out):
    dinv = _dinv_block(degp[...])
    tot = acc[0] + acc[1] + xs[...]
    o = tot * dinv + b[...]
    elu = jnp.where(o > 0.0, o, jnp.exp(o) - 1.0)
    hn = h[...] + 0.3 * elu

    cb = jnp.dot(hn, wcl1T[...], preferred_element_type=jnp.float32)
    cb = (cb + bcl1[...]) * _INV_BN * g_cl[...] + be_cl[...]
    cb = jnp.maximum(cb, 0.0)
    th_out[...] = jnp.dot(cb, wcl2T[...],
                          preferred_element_type=jnp.float32) + bcl2[...]

    r = jnp.maximum(jnp.dot(hn, wr1T[...],
                            preferred_element_type=jnp.float32) + br1[...], 0.0)
    r = jnp.maximum(jnp.dot(r, wr2T[...],
                            preferred_element_type=jnp.float32) + br2[...], 0.0)
    risk_out[...] = _sigmoid(jnp.dot(r, wr3T[...],
                                     preferred_element_type=jnp.float32) + br3[...])

    z = jnp.maximum(jnp.dot(hn, wz1T[...],
                            preferred_element_type=jnp.float32) + bz1[...], 0.0)
    z = jnp.maximum(jnp.dot(z, wz2T[...],
                            preferred_element_type=jnp.float32) + bz2[...], 0.0)
    zd_out[...] = _sigmoid(jnp.dot(z, wz3T[...],
                                   preferred_element_type=jnp.float32) + bz3[...])


def _row_spec(width):
    return pl.BlockSpec((_RB, width), lambda i: (i, 0))


def _full_spec(shape):
    nd = len(shape)
    return pl.BlockSpec(shape, lambda i: (0,) * nd)


_ACC_SPEC = pl.BlockSpec((_NC, _RB, H), lambda i: (0, i, 0))
_DEGP_SPEC = _ACC_SPEC
_GRID = (N // _RB,)


def _tc_input(x, winT, b_in, g_in, be_in, wc0T, degp):
    return pl.pallas_call(
        _tc_input_body,
        grid=_GRID,
        in_specs=[
            _row_spec(H), _full_spec((H, H)), _full_spec((1, H)),
            _full_spec((1, H)), _full_spec((1, H)), _full_spec((H, H)),
            _DEGP_SPEC,
        ],
        out_specs=[_row_spec(H), _row_spec(H)],
        out_shape=[jax.ShapeDtypeStruct((N, H), jnp.float32),
                   jax.ShapeDtypeStruct((N, H), jnp.float32)],
    )(x, winT, b_in, g_in, be_in, wc0T, degp)


def _tc_mid(h, xs, acc, degp, b, wT):
    return pl.pallas_call(
        _tc_mid_body,
        grid=_GRID,
        in_specs=[
            _row_spec(H), _row_spec(H), _ACC_SPEC, _DEGP_SPEC,
            _full_spec((1, H)), _full_spec((H, H)),
        ],
        out_specs=[_row_spec(H), _row_spec(H)],
        out_shape=[jax.ShapeDtypeStruct((N, H), jnp.float32),
                   jax.ShapeDtypeStruct((N, H), jnp.float32)],
    )(h, xs, acc, degp, b, wT)


def _tc_final(h, xs, acc, degp, b, *heads):
    head_specs = [_full_spec(w.shape) for w in heads]
    return pl.pallas_call(
        _tc_final_body,
        grid=_GRID,
        in_specs=[
            _row_spec(H), _row_spec(H), _ACC_SPEC, _DEGP_SPEC,
            _full_spec((1, H)),
        ] + head_specs,
        out_specs=[_row_spec(56), _row_spec(1), _row_spec(1)],
        out_shape=[jax.ShapeDtypeStruct((N, 56), jnp.float32),
                   jax.ShapeDtypeStruct((N, 1), jnp.float32),
                   jax.ShapeDtypeStruct((N, 1), jnp.float32)],
    )(h, xs, acc, degp, b, *heads)


def kernel(x, edge_index, W_in, b_in, g_in, be_in, Wc0, bc0, Wc1, bc1,
           Wc2, bc2, Wcl1, bcl1, g_cl, be_cl, Wcl2, bcl2, Wr1, br1,
           Wr2, br2, Wr3, br3, Wz1, bz1, Wz2, bz2, Wz3, bz3):
    ei = edge_index.reshape(2, _ER, 128).transpose(1, 0, 2)

    ones = jnp.ones((N, H), jnp.float32)
    ei0 = ei.at[:, 0, :].set(jnp.arange(128, dtype=ei.dtype))
    degp = _sc_layer(ones, ei0)

    row = lambda v: v.reshape(1, -1)
    h, xs = _tc_input(x, W_in.T, row(b_in), row(g_in), row(be_in),
                      Wc0.T, degp)

    acc = _sc_layer(xs, ei)
    h, xs = _tc_mid(h, xs, acc, degp, row(bc0), Wc1.T)

    acc = _sc_layer(xs, ei)
    h, xs = _tc_mid(h, xs, acc, degp, row(bc1), Wc2.T)

    acc = _sc_layer(xs, ei)
    threat, risk, zero_day = _tc_final(
        h, xs, acc, degp, row(bc2),
        Wcl1.T, row(bcl1), row(g_cl), row(be_cl), Wcl2.T, row(bcl2),
        Wr1.T, row(br1), Wr2.T, row(br2), Wr3.T, row(br3),
        Wz1.T, row(bz1), Wz2.T, row(bz2), Wz3.T, row(bz3))
    return (threat, risk, zero_day)


# speedup vs baseline: 24.5170x; 1.3865x over previous
import functools
import math

import jax
import jax.numpy as jnp
from jax import lax
from jax.experimental import pallas as pl
from jax.experimental.pallas import tpu as pltpu
from jax.experimental.pallas import tpu_sc as plsc

N = 10000
E = 320000
H = 128
EPS = 1e-5

_NC = 2
_NS = 16
_NW = _NC * _NS
_EPW = E // _NW
_CH = 80
_NCHUNK = _EPW // _CH
_NP = 10240
_RPT = _NP // _NS
_ZR = 64

_INV_BN = 1.0 / math.sqrt(1.0 + EPS)

_mesh = plsc.VectorSubcoreMesh(core_axis_name="c", subcore_axis_name="s")


_ER = E // 128
_FULLC = _ER // _NW
_EXTRA = _ER % _NW


def _sc_layer_body(xs_hbm, ei_hbm, out_hbm, idx, rows, zbuf, acc,
                   sem_i, sem_r, sem_w):
    c = lax.axis_index("c")
    s = lax.axis_index("s")
    wid = c * _NS + s
    nchunk = _FULLC + jnp.where(wid < _EXTRA, 1, 0)

    zeros16 = jnp.zeros((16,), jnp.float32)

    def zfill(i, _):
        for j in range(H // 16):
            zbuf[i, pl.ds(j * 16, 16)] = zeros16
        return ()
    lax.fori_loop(0, _ZR, zfill, ())

    for k in range(_RPT // _ZR):
        pltpu.sync_copy(zbuf, acc.at[pl.ds(s * _RPT + k * _ZR, _ZR)])
    plsc.subcore_barrier()

    def fire_idx(g, q):
        pltpu.async_copy(ei_hbm.at[wid + _NW * g], idx.at[q], sem_i.at[q])

    def wait_idx(q):
        pltpu.make_async_copy(ei_hbm.at[0], idx.at[q], sem_i.at[q]).wait()

    def fire_gather(q, b):
        pltpu.async_copy(xs_hbm.at[idx.at[q, 0]], rows.at[b], sem_r.at[b])

    def wait_gather(q, b):
        pltpu.make_async_copy(xs_hbm.at[idx.at[q, 0]], rows.at[b],
                              sem_r.at[b]).wait()

    def fire_scatter(q, b):
        pltpu.async_copy(rows.at[b], acc.at[idx.at[q, 1]], sem_w.at[b],
                         add=True)

    def wait_scatter(q, b):
        pltpu.make_async_copy(rows.at[b], acc.at[idx.at[q, 1]],
                              sem_w.at[b]).wait()

    fire_idx(0, 0)

    @pl.when(nchunk > 1)
    def _():
        fire_idx(1, 1)
    wait_idx(0)
    fire_gather(0, 0)

    def body(g, _):
        b = lax.rem(g, 2)
        nb = lax.rem(g + 1, 2)
        q = lax.rem(g, 4)
        q1 = lax.rem(g + 1, 4)
        q2 = lax.rem(g + 2, 4)
        qm1 = lax.rem(g + 3, 4)

        @pl.when(g + 1 < nchunk)
        def _():
            wait_idx(q1)

            @pl.when(g >= 1)
            def _():
                wait_scatter(qm1, nb)
            fire_gather(q1, nb)
        wait_gather(q, b)
        fire_scatter(q, b)

        @pl.when(g + 2 < nchunk)
        def _():
            fire_idx(g + 2, q2)
        return ()
    lax.fori_loop(0, nchunk, body, ())

    @pl.when(nchunk >= 2)
    def _():
        wait_scatter(lax.rem(nchunk + 2, 4), lax.rem(nchunk, 2))
    wait_scatter(lax.rem(nchunk + 3, 4), lax.rem(nchunk + 1, 2))

    plsc.subcore_barrier()
    pltpu.sync_copy(acc.at[pl.ds(s * _RPT, _RPT)],
                    out_hbm.at[c].at[pl.ds(s * _RPT, _RPT)])


_sc_layer = pl.kernel(
    _sc_layer_body,
    mesh=_mesh,
    out_type=jax.ShapeDtypeStruct((_NC, _NP, H), jnp.float32),
    scratch_types=[
        pltpu.VMEM((4, 2, 128), jnp.int32),
        pltpu.VMEM((2, 128, H), jnp.float32),
        pltpu.VMEM((_ZR, H), jnp.float32),
        pltpu.VMEM_SHARED((_NP, H), jnp.float32),
        pltpu.SemaphoreType.DMA((4,)),
        pltpu.SemaphoreType.DMA((2,)),
        pltpu.SemaphoreType.DMA((2,)),
    ],
)


_RB = 1000


def _dinv_block(degp):
    deg = degp[0, :, 0:1] + degp[1, :, 0:1] + 1.0
    return lax.rsqrt(deg)


def _tc_input_body(x, winT, b_in, g_in, be_in, wc0T, degp, h_out, xs_out):
    hv = jnp.dot(x[...], winT[...], preferred_element_type=jnp.float32)
    hv = (hv + b_in[...]) * _INV_BN * g_in[...] + be_in[...]
    hv = jnp.maximum(hv, 0.0)
    h_out[...] = hv
    dinv = _dinv_block(degp[...])
    xs_out[...] = jnp.dot(hv, wc0T[...],
                          preferred_element_type=jnp.float32) * dinv


def _tc_mid_body(h, xs, acc, degp, b, wT, h_out, xs_out):
    dinv = _dinv_block(degp[...])
    tot = acc[0] + acc[1] + xs[...]
    o = tot * dinv + b[...]
    elu = jnp.where(o > 0.0, o, jnp.exp(o) - 1.0)
    hn = h[...] + 0.3 * elu
    h_out[...] = hn
    xs_out[...] = jnp.dot(hn, wT[...],
                          preferred_element_type=jnp.float32) * dinv


def _sigmoid(t):
    return 1.0 / (1.0 + jnp.exp(-t))


def _tc_final_body(h, xs, acc, degp, b,
                   wcl1T, bcl1, g_cl, be_cl, wcl2T, bcl2,
                   wr1T, br1, wr2T, br2, wr3T, br3,
                   wz1T, bz1, wz2T, bz2, wz3T, bz3,
                   th_out, risk_out, zd_out):
    dinv = _dinv_block(degp[...])
    tot = acc[0] + acc[1] + xs[...]
    o = tot * dinv + b[...]
    elu = jnp.where(o > 0.0, o, jnp.exp(o) - 1.0)
    hn = h[...] + 0.3 * elu

    cb = jnp.dot(hn, wcl1T[...], preferred_element_type=jnp.float32)
    cb = (cb + bcl1[...]) * _INV_BN * g_cl[...] + be_cl[...]
    cb = jnp.maximum(cb, 0.0)
    th_out[...] = jnp.dot(cb, wcl2T[...],
                          preferred_element_type=jnp.float32) + bcl2[...]

    r = jnp.maximum(jnp.dot(hn, wr1T[...],
                            preferred_element_type=jnp.float32) + br1[...], 0.0)
    r = jnp.maximum(jnp.dot(r, wr2T[...],
                            preferred_element_type=jnp.float32) + br2[...], 0.0)
    risk_out[...] = _sigmoid(jnp.dot(r, wr3T[...],
                                     preferred_element_type=jnp.float32) + br3[...])

    z = jnp.maximum(jnp.dot(hn, wz1T[...],
                            preferred_element_type=jnp.float32) + bz1[...], 0.0)
    z = jnp.maximum(jnp.dot(z, wz2T[...],
                            preferred_element_type=jnp.float32) + bz2[...], 0.0)
    zd_out[...] = _sigmoid(jnp.dot(z, wz3T[...],
                                   preferred_element_type=jnp.float32) + bz3[...])


def _row_spec(width):
    return pl.BlockSpec((_RB, width), lambda i: (i, 0))


def _full_spec(shape):
    nd = len(shape)
    return pl.BlockSpec(shape, lambda i: (0,) * nd)


_ACC_SPEC = pl.BlockSpec((_NC, _RB, H), lambda i: (0, i, 0))
_DEGP_SPEC = _ACC_SPEC
_GRID = (N // _RB,)


def _tc_input(x, winT, b_in, g_in, be_in, wc0T, degp):
    return pl.pallas_call(
        _tc_input_body,
        grid=_GRID,
        in_specs=[
            _row_spec(H), _full_spec((H, H)), _full_spec((1, H)),
            _full_spec((1, H)), _full_spec((1, H)), _full_spec((H, H)),
            _DEGP_SPEC,
        ],
        out_specs=[_row_spec(H), _row_spec(H)],
        out_shape=[jax.ShapeDtypeStruct((N, H), jnp.float32),
                   jax.ShapeDtypeStruct((N, H), jnp.float32)],
    )(x, winT, b_in, g_in, be_in, wc0T, degp)


def _tc_mid(h, xs, acc, degp, b, wT):
    return pl.pallas_call(
        _tc_mid_body,
        grid=_GRID,
        in_specs=[
            _row_spec(H), _row_spec(H), _ACC_SPEC, _DEGP_SPEC,
            _full_spec((1, H)), _full_spec((H, H)),
        ],
        out_specs=[_row_spec(H), _row_spec(H)],
        out_shape=[jax.ShapeDtypeStruct((N, H), jnp.float32),
                   jax.ShapeDtypeStruct((N, H), jnp.float32)],
    )(h, xs, acc, degp, b, wT)


def _tc_final(h, xs, acc, degp, b, *heads):
    head_specs = [_full_spec(w.shape) for w in heads]
    return pl.pallas_call(
        _tc_final_body,
        grid=_GRID,
        in_specs=[
            _row_spec(H), _row_spec(H), _ACC_SPEC, _DEGP_SPEC,
            _full_spec((1, H)),
        ] + head_specs,
        out_specs=[_row_spec(56), _row_spec(1), _row_spec(1)],
        out_shape=[jax.ShapeDtypeStruct((N, 56), jnp.float32),
                   jax.ShapeDtypeStruct((N, 1), jnp.float32),
                   jax.ShapeDtypeStruct((N, 1), jnp.float32)],
    )(h, xs, acc, degp, b, *heads)


def kernel(x, edge_index, W_in, b_in, g_in, be_in, Wc0, bc0, Wc1, bc1,
           Wc2, bc2, Wcl1, bcl1, g_cl, be_cl, Wcl2, bcl2, Wr1, br1,
           Wr2, br2, Wr3, br3, Wz1, bz1, Wz2, bz2, Wz3, bz3):
    ei = edge_index.reshape(2, _ER, 128).transpose(1, 0, 2)

    ones = jnp.ones((N, H), jnp.float32)
    degp = _sc_layer(ones, ei)

    row = lambda v: v.reshape(1, -1)
    h, xs = _tc_input(x, W_in.T, row(b_in), row(g_in), row(be_in),
                      Wc0.T, degp)

    acc = _sc_layer(xs, ei)
    h, xs = _tc_mid(h, xs, acc, degp, row(bc0), Wc1.T)

    acc = _sc_layer(xs, ei)
    h, xs = _tc_mid(h, xs, acc, degp, row(bc1), Wc2.T)

    acc = _sc_layer(xs, ei)
    threat, risk, zero_day = _tc_final(
        h, xs, acc, degp, row(bc2),
        Wcl1.T, row(bcl1), row(g_cl), row(be_cl), Wcl2.T, row(bcl2),
        Wr1.T, row(br1), Wr2.T, row(br2), Wr3.T, row(br3),
        Wz1.T, row(bz1), Wz2.T, row(bz2), Wz3.T, row(bz3))
    return (threat, risk, zero_day)


# speedup vs baseline: 24.9486x; 1.0176x over previous
import functools
import math

import jax
import jax.numpy as jnp
from jax import lax
from jax.experimental import pallas as pl
from jax.experimental.pallas import tpu as pltpu
from jax.experimental.pallas import tpu_sc as plsc

N = 10000
E = 320000
H = 128
EPS = 1e-5

_NC = 2
_NS = 16
_NW = _NC * _NS
_EPW = E // _NW
_CH = 80
_NCHUNK = _EPW // _CH
_NP = 10240
_RPT = _NP // _NS
_ZR = 64

_INV_BN = 1.0 / math.sqrt(1.0 + EPS)

_mesh = plsc.VectorSubcoreMesh(core_axis_name="c", subcore_axis_name="s")


_ER = E // 128
_FULLC = _ER // _NW
_EXTRA = _ER % _NW


def _sc_layer_body(xs_hbm, ei_hbm, out_hbm, idx, rows, zbuf, acc,
                   sem_i, sem_r, sem_w):
    c = lax.axis_index("c")
    s = lax.axis_index("s")
    wid = c * _NS + s
    nchunk = _FULLC + jnp.where(wid < _EXTRA, 1, 0)

    def fire_idx(g, q):
        pltpu.async_copy(ei_hbm.at[wid + _NW * g], idx.at[q], sem_i.at[q])

    def wait_idx(q):
        pltpu.make_async_copy(ei_hbm.at[0], idx.at[q], sem_i.at[q]).wait()

    def fire_gather(q, b):
        pltpu.async_copy(xs_hbm.at[idx.at[q, 0]], rows.at[b], sem_r.at[b])

    def wait_gather(q, b):
        pltpu.make_async_copy(xs_hbm.at[idx.at[q, 0]], rows.at[b],
                              sem_r.at[b]).wait()

    def fire_scatter(q, b):
        pltpu.async_copy(rows.at[b], acc.at[idx.at[q, 1]], sem_w.at[b],
                         add=True)

    def wait_scatter(q, b):
        pltpu.make_async_copy(rows.at[b], acc.at[idx.at[q, 1]],
                              sem_w.at[b]).wait()

    fire_idx(0, 0)

    @pl.when(nchunk > 1)
    def _():
        fire_idx(1, 1)
    wait_idx(0)
    fire_gather(0, 0)

    zeros16 = jnp.zeros((16,), jnp.float32)

    def zfill(i, _):
        for j in range(H // 16):
            zbuf[i, pl.ds(j * 16, 16)] = zeros16
        return ()
    lax.fori_loop(0, _ZR, zfill, ())

    for k in range(_RPT // _ZR):
        pltpu.sync_copy(zbuf, acc.at[pl.ds(s * _RPT + k * _ZR, _ZR)])
    plsc.subcore_barrier()

    def body(g, _):
        b = lax.rem(g, 2)
        nb = lax.rem(g + 1, 2)
        q = lax.rem(g, 4)
        q1 = lax.rem(g + 1, 4)
        q2 = lax.rem(g + 2, 4)
        qm1 = lax.rem(g + 3, 4)

        @pl.when(g + 1 < nchunk)
        def _():
            wait_idx(q1)

            @pl.when(g >= 1)
            def _():
                wait_scatter(qm1, nb)
            fire_gather(q1, nb)
        wait_gather(q, b)
        fire_scatter(q, b)

        @pl.when(g + 2 < nchunk)
        def _():
            fire_idx(g + 2, q2)
        return ()
    lax.fori_loop(0, nchunk, body, ())

    @pl.when(nchunk >= 2)
    def _():
        wait_scatter(lax.rem(nchunk + 2, 4), lax.rem(nchunk, 2))
    wait_scatter(lax.rem(nchunk + 3, 4), lax.rem(nchunk + 1, 2))

    plsc.subcore_barrier()
    pltpu.sync_copy(acc.at[pl.ds(s * _RPT, _RPT)],
                    out_hbm.at[c].at[pl.ds(s * _RPT, _RPT)])


_sc_layer = pl.kernel(
    _sc_layer_body,
    mesh=_mesh,
    out_type=jax.ShapeDtypeStruct((_NC, _NP, H), jnp.float32),
    scratch_types=[
        pltpu.VMEM((4, 2, 128), jnp.int32),
        pltpu.VMEM((2, 128, H), jnp.float32),
        pltpu.VMEM((_ZR, H), jnp.float32),
        pltpu.VMEM_SHARED((_NP, H), jnp.float32),
        pltpu.SemaphoreType.DMA((4,)),
        pltpu.SemaphoreType.DMA((2,)),
        pltpu.SemaphoreType.DMA((2,)),
    ],
)


_RB = 1000


def _dinv_block(degp):
    deg = degp[0, :, 0:1] + degp[1, :, 0:1] + 1.0
    return lax.rsqrt(deg)


def _tc_input_body(x, winT, b_in, g_in, be_in, wc0T, degp,
                   h_out, xs_out, dinv_out):
    hv = jnp.dot(x[...], winT[...], preferred_element_type=jnp.float32)
    hv = (hv + b_in[...]) * _INV_BN * g_in[...] + be_in[...]
    hv = jnp.maximum(hv, 0.0)
    h_out[...] = hv
    dinv = _dinv_block(degp[...])
    dinv_out[...] = dinv
    xs_out[...] = jnp.dot(hv, wc0T[...],
                          preferred_element_type=jnp.float32) * dinv


def _tc_mid_body(h, xs, acc, dinv_in, b, wT, h_out, xs_out):
    dinv = dinv_in[...]
    tot = acc[0] + acc[1] + xs[...]
    o = tot * dinv + b[...]
    elu = jnp.where(o > 0.0, o, jnp.exp(o) - 1.0)
    hn = h[...] + 0.3 * elu
    h_out[...] = hn
    xs_out[...] = jnp.dot(hn, wT[...],
                          preferred_element_type=jnp.float32) * dinv


def _sigmoid(t):
    return 1.0 / (1.0 + jnp.exp(-t))


def _tc_final_body(h, xs, acc, dinv_in, b,
                   wcl1T, bcl1, g_cl, be_cl, wcl2T, bcl2,
                   wr1T, br1, wr2T, br2, wr3T, br3,
                   wz1T, bz1, wz2T, bz2, wz3T, bz3,
                   th_out, risk_out, zd_out):
    dinv = dinv_in[...]
    tot = acc[0] + acc[1] + xs[...]
    o = tot * dinv + b[...]
    elu = jnp.where(o > 0.0, o, jnp.exp(o) - 1.0)
    hn = h[...] + 0.3 * elu

    cb = jnp.dot(hn, wcl1T[...], preferred_element_type=jnp.float32)
    cb = (cb + bcl1[...]) * _INV_BN * g_cl[...] + be_cl[...]
    cb = jnp.maximum(cb, 0.0)
    th_out[...] = jnp.dot(cb, wcl2T[...],
                          preferred_element_type=jnp.float32) + bcl2[...]

    r = jnp.maximum(jnp.dot(hn, wr1T[...],
                            preferred_element_type=jnp.float32) + br1[...], 0.0)
    r = jnp.maximum(jnp.dot(r, wr2T[...],
                            preferred_element_type=jnp.float32) + br2[...], 0.0)
    risk_out[...] = _sigmoid(jnp.dot(r, wr3T[...],
                                     preferred_element_type=jnp.float32) + br3[...])

    z = jnp.maximum(jnp.dot(hn, wz1T[...],
                            preferred_element_type=jnp.float32) + bz1[...], 0.0)
    z = jnp.maximum(jnp.dot(z, wz2T[...],
                            preferred_element_type=jnp.float32) + bz2[...], 0.0)
    zd_out[...] = _sigmoid(jnp.dot(z, wz3T[...],
                                   preferred_element_type=jnp.float32) + bz3[...])


def _row_spec(width):
    return pl.BlockSpec((_RB, width), lambda i: (i, 0))


def _full_spec(shape):
    nd = len(shape)
    return pl.BlockSpec(shape, lambda i: (0,) * nd)


_ACC_SPEC = pl.BlockSpec((_NC, _RB, H), lambda i: (0, i, 0))
_DEGP_SPEC = _ACC_SPEC
_GRID = (N // _RB,)


def _tc_input(x, winT, b_in, g_in, be_in, wc0T, degp):
    return pl.pallas_call(
        _tc_input_body,
        grid=_GRID,
        in_specs=[
            _row_spec(H), _full_spec((H, H)), _full_spec((1, H)),
            _full_spec((1, H)), _full_spec((1, H)), _full_spec((H, H)),
            _DEGP_SPEC,
        ],
        out_specs=[_row_spec(H), _row_spec(H), _row_spec(1)],
        out_shape=[jax.ShapeDtypeStruct((N, H), jnp.float32),
                   jax.ShapeDtypeStruct((N, H), jnp.float32),
                   jax.ShapeDtypeStruct((N, 1), jnp.float32)],
    )(x, winT, b_in, g_in, be_in, wc0T, degp)


def _tc_mid(h, xs, acc, dinv, b, wT):
    return pl.pallas_call(
        _tc_mid_body,
        grid=_GRID,
        in_specs=[
            _row_spec(H), _row_spec(H), _ACC_SPEC, _row_spec(1),
            _full_spec((1, H)), _full_spec((H, H)),
        ],
        out_specs=[_row_spec(H), _row_spec(H)],
        out_shape=[jax.ShapeDtypeStruct((N, H), jnp.float32),
                   jax.ShapeDtypeStruct((N, H), jnp.float32)],
    )(h, xs, acc, dinv, b, wT)


def _tc_final(h, xs, acc, dinv, b, *heads):
    head_specs = [_full_spec(w.shape) for w in heads]
    return pl.pallas_call(
        _tc_final_body,
        grid=_GRID,
        in_specs=[
            _row_spec(H), _row_spec(H), _ACC_SPEC, _row_spec(1),
            _full_spec((1, H)),
        ] + head_specs,
        out_specs=[_row_spec(56), _row_spec(1), _row_spec(1)],
        out_shape=[jax.ShapeDtypeStruct((N, 56), jnp.float32),
                   jax.ShapeDtypeStruct((N, 1), jnp.float32),
                   jax.ShapeDtypeStruct((N, 1), jnp.float32)],
    )(h, xs, acc, dinv, b, *heads)


def kernel(x, edge_index, W_in, b_in, g_in, be_in, Wc0, bc0, Wc1, bc1,
           Wc2, bc2, Wcl1, bcl1, g_cl, be_cl, Wcl2, bcl2, Wr1, br1,
           Wr2, br2, Wr3, br3, Wz1, bz1, Wz2, bz2, Wz3, bz3):
    ei = edge_index.reshape(2, _ER, 128).transpose(1, 0, 2)

    ones = jnp.ones((N, H), jnp.float32)
    degp = _sc_layer(ones, ei)

    row = lambda v: v.reshape(1, -1)
    h, xs, dinv = _tc_input(x, W_in.T, row(b_in), row(g_in), row(be_in),
                            Wc0.T, degp)

    acc = _sc_layer(xs, ei)
    h, xs = _tc_mid(h, xs, acc, dinv, row(bc0), Wc1.T)

    acc = _sc_layer(xs, ei)
    h, xs = _tc_mid(h, xs, acc, dinv, row(bc1), Wc2.T)

    acc = _sc_layer(xs, ei)
    threat, risk, zero_day = _tc_final(
        h, xs, acc, dinv, row(bc2),
        Wcl1.T, row(bcl1), row(g_cl), row(be_cl), Wcl2.T, row(bcl2),
        Wr1.T, row(br1), Wr2.T, row(br2), Wr3.T, row(br3),
        Wz1.T, row(bz1), Wz2.T, row(bz2), Wz3.T, row(bz3))
    return (threat, risk, zero_day)


# speedup vs baseline: 24.9780x; 1.0012x over previous
import math

import jax
import jax.numpy as jnp
from jax import lax
from jax.experimental import pallas as pl
from jax.experimental.pallas import tpu as pltpu
from jax.experimental.pallas import tpu_sc as plsc

N = 10000
E = 320000
H = 128
EPS = 1e-5

_NC = 2
_NS = 16
_NW = _NC * _NS
_EPW = E // _NW
_CH = 80
_NCHUNK = _EPW // _CH
_NP = 10240
_RPT = _NP // _NS
_ZR = 64

_INV_BN = 1.0 / math.sqrt(1.0 + EPS)

_mesh = plsc.VectorSubcoreMesh(core_axis_name="c", subcore_axis_name="s")


_ER = E // 128
_FULLC = _ER // _NW
_EXTRA = _ER % _NW


def _sc_layer_body(xs_hbm, ei_hbm, out_hbm, idx, rows, zbuf, acc,
                   sem_i, sem_r, sem_w):
    c = lax.axis_index("c")
    s = lax.axis_index("s")
    wid = c * _NS + s
    nchunk = _FULLC + jnp.where(wid < _EXTRA, 1, 0)

    def fire_idx(g, q):
        pltpu.async_copy(ei_hbm.at[wid + _NW * g], idx.at[q], sem_i.at[q])

    def wait_idx(q):
        pltpu.make_async_copy(ei_hbm.at[0], idx.at[q], sem_i.at[q]).wait()

    def fire_gather(q, b):
        pltpu.async_copy(xs_hbm.at[idx.at[q, 0]], rows.at[b], sem_r.at[b])

    def wait_gather(q, b):
        pltpu.make_async_copy(xs_hbm.at[idx.at[q, 0]], rows.at[b],
                              sem_r.at[b]).wait()

    def fire_scatter(q, b):
        pltpu.async_copy(rows.at[b], acc.at[idx.at[q, 1]], sem_w.at[b],
                         add=True)

    def wait_scatter(q, b):
        pltpu.make_async_copy(rows.at[b], acc.at[idx.at[q, 1]],
                              sem_w.at[b]).wait()

    fire_idx(0, 0)

    @pl.when(nchunk > 1)
    def _():
        fire_idx(1, 1)
    wait_idx(0)
    fire_gather(0, 0)

    zeros16 = jnp.zeros((16,), jnp.float32)

    def zfill(i, _):
        for j in range(H // 16):
            zbuf[i, pl.ds(j * 16, 16)] = zeros16
        return ()
    lax.fori_loop(0, _ZR, zfill, ())

    for k in range(_RPT // _ZR):
        pltpu.sync_copy(zbuf, acc.at[pl.ds(s * _RPT + k * _ZR, _ZR)])
    plsc.subcore_barrier()

    def body(g, _):
        b = lax.rem(g, 2)
        nb = lax.rem(g + 1, 2)
        q = lax.rem(g, 4)
        q1 = lax.rem(g + 1, 4)
        q2 = lax.rem(g + 2, 4)
        qm1 = lax.rem(g + 3, 4)

        @pl.when(g + 1 < nchunk)
        def _():
            wait_idx(q1)

            @pl.when(g >= 1)
            def _():
                wait_scatter(qm1, nb)
            fire_gather(q1, nb)
        wait_gather(q, b)
        fire_scatter(q, b)

        @pl.when(g + 2 < nchunk)
        def _():
            fire_idx(g + 2, q2)
        return ()
    lax.fori_loop(0, nchunk, body, ())

    @pl.when(nchunk >= 2)
    def _():
        wait_scatter(lax.rem(nchunk + 2, 4), lax.rem(nchunk, 2))
    wait_scatter(lax.rem(nchunk + 3, 4), lax.rem(nchunk + 1, 2))

    plsc.subcore_barrier()
    pltpu.sync_copy(acc.at[pl.ds(s * _RPT, _RPT)],
                    out_hbm.at[c].at[pl.ds(s * _RPT, _RPT)])


_sc_layer = pl.kernel(
    _sc_layer_body,
    mesh=_mesh,
    out_type=jax.ShapeDtypeStruct((_NC, _NP, H), jnp.float32),
    scratch_types=[
        pltpu.VMEM((4, 2, 128), jnp.int32),
        pltpu.VMEM((2, 128, H), jnp.float32),
        pltpu.VMEM((_ZR, H), jnp.float32),
        pltpu.VMEM_SHARED((_NP, H), jnp.float32),
        pltpu.SemaphoreType.DMA((4,)),
        pltpu.SemaphoreType.DMA((2,)),
        pltpu.SemaphoreType.DMA((2,)),
    ],
)


_RB = 1000


def _dinv_block(degp):
    deg = degp[0, :, 0:1] + degp[1, :, 0:1] + 1.0
    return lax.rsqrt(deg)


def _tc_input_body(x, winT, b_in, g_in, be_in, wc0T, degp,
                   h_out, xs_out, dinv_out):
    hv = jnp.dot(x[...], winT[...], preferred_element_type=jnp.float32)
    hv = (hv + b_in[...]) * _INV_BN * g_in[...] + be_in[...]
    hv = jnp.maximum(hv, 0.0)
    h_out[...] = hv
    dinv = _dinv_block(degp[...])
    dinv_out[...] = dinv
    xs_out[...] = jnp.dot(hv, wc0T[...],
                          preferred_element_type=jnp.float32) * dinv


def _tc_mid_body(h, xs, acc, dinv_in, b, wT, h_out, xs_out):
    dinv = dinv_in[...]
    tot = acc[0] + acc[1] + xs[...]
    o = tot * dinv + b[...]
    elu = jnp.where(o > 0.0, o, jnp.exp(o) - 1.0)
    hn = h[...] + 0.3 * elu
    h_out[...] = hn
    xs_out[...] = jnp.dot(hn, wT[...],
                          preferred_element_type=jnp.float32) * dinv


def _sigmoid(t):
    return 1.0 / (1.0 + jnp.exp(-t))


def _tc_final_body(h, xs, acc, dinv_in, b,
                   wcl1T, bcl1, g_cl, be_cl, wcl2T, bcl2,
                   wr1T, br1, wr2T, br2, wr3T, br3,
                   wz1T, bz1, wz2T, bz2, wz3T, bz3,
                   th_out, risk_out, zd_out):
    dinv = dinv_in[...]
    tot = acc[0] + acc[1] + xs[...]
    o = tot * dinv + b[...]
    elu = jnp.where(o > 0.0, o, jnp.exp(o) - 1.0)
    hn = h[...] + 0.3 * elu

    cb = jnp.dot(hn, wcl1T[...], preferred_element_type=jnp.float32)
    cb = (cb + bcl1[...]) * _INV_BN * g_cl[...] + be_cl[...]
    cb = jnp.maximum(cb, 0.0)
    th_out[...] = jnp.dot(cb, wcl2T[...],
                          preferred_element_type=jnp.float32) + bcl2[...]

    r = jnp.maximum(jnp.dot(hn, wr1T[...],
                            preferred_element_type=jnp.float32) + br1[...], 0.0)
    r = jnp.maximum(jnp.dot(r, wr2T[...],
                            preferred_element_type=jnp.float32) + br2[...], 0.0)
    risk_out[...] = _sigmoid(jnp.dot(r, wr3T[...],
                                     preferred_element_type=jnp.float32) + br3[...])

    z = jnp.maximum(jnp.dot(hn, wz1T[...],
                            preferred_element_type=jnp.float32) + bz1[...], 0.0)
    z = jnp.maximum(jnp.dot(z, wz2T[...],
                            preferred_element_type=jnp.float32) + bz2[...], 0.0)
    zd_out[...] = _sigmoid(jnp.dot(z, wz3T[...],
                                   preferred_element_type=jnp.float32) + bz3[...])


def _row_spec(width):
    return pl.BlockSpec((_RB, width), lambda i: (i, 0))


def _full_spec(shape):
    nd = len(shape)
    return pl.BlockSpec(shape, lambda i: (0,) * nd)


_ACC_SPEC = pl.BlockSpec((_NC, _RB, H), lambda i: (0, i, 0))
_DEGP_SPEC = _ACC_SPEC
_GRID = (N // _RB,)


def _tc_input(x, winT, b_in, g_in, be_in, wc0T, degp):
    return pl.pallas_call(
        _tc_input_body,
        grid=_GRID,
        in_specs=[
            _row_spec(H), _full_spec((H, H)), _full_spec((1, H)),
            _full_spec((1, H)), _full_spec((1, H)), _full_spec((H, H)),
            _DEGP_SPEC,
        ],
        out_specs=[_row_spec(H), _row_spec(H), _row_spec(1)],
        out_shape=[jax.ShapeDtypeStruct((N, H), jnp.float32),
                   jax.ShapeDtypeStruct((N, H), jnp.float32),
                   jax.ShapeDtypeStruct((N, 1), jnp.float32)],
    )(x, winT, b_in, g_in, be_in, wc0T, degp)


def _tc_mid(h, xs, acc, dinv, b, wT):
    return pl.pallas_call(
        _tc_mid_body,
        grid=_GRID,
        in_specs=[
            _row_spec(H), _row_spec(H), _ACC_SPEC, _row_spec(1),
            _full_spec((1, H)), _full_spec((H, H)),
        ],
        out_specs=[_row_spec(H), _row_spec(H)],
        out_shape=[jax.ShapeDtypeStruct((N, H), jnp.float32),
                   jax.ShapeDtypeStruct((N, H), jnp.float32)],
    )(h, xs, acc, dinv, b, wT)


def _tc_final(h, xs, acc, dinv, b, *heads):
    head_specs = [_full_spec(w.shape) for w in heads]
    return pl.pallas_call(
        _tc_final_body,
        grid=_GRID,
        in_specs=[
            _row_spec(H), _row_spec(H), _ACC_SPEC, _row_spec(1),
            _full_spec((1, H)),
        ] + head_specs,
        out_specs=[_row_spec(56), _row_spec(1), _row_spec(1)],
        out_shape=[jax.ShapeDtypeStruct((N, 56), jnp.float32),
                   jax.ShapeDtypeStruct((N, 1), jnp.float32),
                   jax.ShapeDtypeStruct((N, 1), jnp.float32)],
    )(h, xs, acc, dinv, b, *heads)


def kernel(x, edge_index, W_in, b_in, g_in, be_in, Wc0, bc0, Wc1, bc1,
           Wc2, bc2, Wcl1, bcl1, g_cl, be_cl, Wcl2, bcl2, Wr1, br1,
           Wr2, br2, Wr3, br3, Wz1, bz1, Wz2, bz2, Wz3, bz3):
    ei = edge_index.reshape(2, _ER, 128).transpose(1, 0, 2)

    ones = jnp.ones((N, H), jnp.float32)
    degp = _sc_layer(ones, ei)

    row = lambda v: v.reshape(1, -1)
    h, xs, dinv = _tc_input(x, W_in.T, row(b_in), row(g_in), row(be_in),
                            Wc0.T, degp)

    acc = _sc_layer(xs, ei)
    h, xs = _tc_mid(h, xs, acc, dinv, row(bc0), Wc1.T)

    acc = _sc_layer(xs, ei)
    h, xs = _tc_mid(h, xs, acc, dinv, row(bc1), Wc2.T)

    acc = _sc_layer(xs, ei)
    threat, risk, zero_day = _tc_final(
        h, xs, acc, dinv, row(bc2),
        Wcl1.T, row(bcl1), row(g_cl), row(be_cl), Wcl2.T, row(bcl2),
        Wr1.T, row(br1), Wr2.T, row(br2), Wr3.T, row(br3),
        Wz1.T, row(bz1), Wz2.T, row(bz2), Wz3.T, row(bz3))
    return (threat, risk, zero_day)
